# baseline (speedup 1.0000x reference)
import functools
import jax
import jax.numpy as jnp
from jax import lax
from jax.experimental import pallas as pl
from jax.experimental.pallas import tpu as pltpu
from jax.experimental.pallas import tpu_sc as plsc

NC, NS, L = 2, 16, 16
NW = NC * NS

CH = 32


def _make_sc(total_rows, seq, dim):
    rows_per_w = total_rows // NW
    n_chunks = rows_per_w // CH
    cw = CH * dim
    mesh = plsc.VectorSubcoreMesh(core_axis_name="c", subcore_axis_name="s")

    @functools.partial(
        pl.kernel,
        mesh=mesh,
        out_type=jax.ShapeDtypeStruct((total_rows * dim,), jnp.float32),
        scratch_types=[
            pltpu.VMEM((cw,), jnp.float32),
            pltpu.VMEM((cw,), jnp.float32),
        ],
    )
    def k(in_hbm, pos_hbm, out_hbm, buf, pbuf):
        wid = lax.axis_index("s") * NC + lax.axis_index("c")
        row0 = wid * rows_per_w
        pos0 = lax.rem(row0, seq)

        def chunk(g, carry):
            r = (row0 + g * CH) * dim
            p = (pos0 + g * CH) * dim
            pltpu.sync_copy(in_hbm.at[pl.ds(r, cw)], buf)
            pltpu.sync_copy(pos_hbm.at[pl.ds(p, cw)], pbuf)

            @plsc.parallel_loop(0, cw, step=L, unroll=8)
            def add_body(i):
                buf[pl.ds(i, L)] = buf[pl.ds(i, L)] + pbuf[pl.ds(i, L)]

            pltpu.sync_copy(buf, out_hbm.at[pl.ds(r, cw)])
            return carry

        lax.fori_loop(0, n_chunks, chunk, None)

    return k


def kernel(inputs, pos_table):
    b, s, d = inputs.shape
    flat = inputs.reshape(b * s * d)
    out = _make_sc(b * s, s, d)(flat, pos_table.reshape(s * d))
    return out.reshape(b, s, d)


# speedup vs baseline: 1.2463x; 1.2463x over previous
import functools
import jax
import jax.numpy as jnp
from jax import lax
from jax.experimental import pallas as pl
from jax.experimental.pallas import tpu as pltpu
from jax.experimental.pallas import tpu_sc as plsc

NC, NS, L = 2, 16, 16
NW = NC * NS

CH = 16
NBUF = 2


def _make_sc(total_rows, seq, dim):
    rows_per_w = total_rows // NW
    n_chunks = rows_per_w // CH
    cw = CH * dim
    mesh = plsc.VectorSubcoreMesh(core_axis_name="c", subcore_axis_name="s")

    @functools.partial(
        pl.kernel,
        mesh=mesh,
        out_type=jax.ShapeDtypeStruct((total_rows * dim,), jnp.float32),
        scratch_types=[
            [pltpu.VMEM((cw,), jnp.float32) for _ in range(NBUF)],
            [pltpu.VMEM((cw,), jnp.float32) for _ in range(NBUF)],
            [pltpu.VMEM((cw,), jnp.float32) for _ in range(NBUF)],
            [pltpu.SemaphoreType.DMA for _ in range(NBUF)],
            [pltpu.SemaphoreType.DMA for _ in range(NBUF)],
        ],
    )
    def k(in_hbm, pos_hbm, out_hbm, bufs, pbufs, obufs, in_sems, out_sems):
        wid = lax.axis_index("s") * NC + lax.axis_index("c")
        row0 = wid * rows_per_w
        pos0 = lax.rem(row0, seq)

        def start_in(g, b):
            r = (row0 + g * CH) * dim
            p = (pos0 + g * CH) * dim
            pltpu.async_copy(in_hbm.at[pl.ds(r, cw)], bufs[b], in_sems[b])
            pltpu.async_copy(pos_hbm.at[pl.ds(p, cw)], pbufs[b], in_sems[b])

        def wait_in(b):
            pltpu.make_async_copy(in_hbm.at[pl.ds(0, cw)], bufs[b], in_sems[b]).wait()
            pltpu.make_async_copy(pos_hbm.at[pl.ds(0, cw)], pbufs[b], in_sems[b]).wait()

        def start_out(g, b):
            r = (row0 + g * CH) * dim
            pltpu.async_copy(obufs[b], out_hbm.at[pl.ds(r, cw)], out_sems[b])

        def wait_out(b):
            pltpu.make_async_copy(obufs[b], out_hbm.at[pl.ds(0, cw)], out_sems[b]).wait()

        for b in range(NBUF):
            start_in(b, b)

        def step(g, b):
            wait_in(b)

            @pl.when(g >= NBUF)
            def _():
                wait_out(b)

            @plsc.parallel_loop(0, cw, step=L, unroll=8)
            def add_body(i):
                obufs[b][pl.ds(i, L)] = bufs[b][pl.ds(i, L)] + pbufs[b][pl.ds(i, L)]

            start_out(g, b)

            @pl.when(g + NBUF < n_chunks)
            def _():
                start_in(g + NBUF, b)

        def outer(go, carry):
            g = go * NBUF
            for b in range(NBUF):
                step(g + b, b)
            return carry

        lax.fori_loop(0, n_chunks // NBUF, outer, None)
        for b in range(NBUF):
            wait_out(b)

    return k


def kernel(inputs, pos_table):
    b, s, d = inputs.shape
    flat = inputs.reshape(b * s * d)
    out = _make_sc(b * s, s, d)(flat, pos_table.reshape(s * d))
    return out.reshape(b, s, d)


# speedup vs baseline: 1.3501x; 1.0832x over previous
import functools
import jax
import jax.numpy as jnp
from jax import lax
from jax.experimental import pallas as pl
from jax.experimental.pallas import tpu as pltpu
from jax.experimental.pallas import tpu_sc as plsc

NC, NS, L = 2, 16, 16
NW = NC * NS

CH = 16
NBUF = 2

SEQ_BLOCK = 512


def _make_sc(total_rows, seq, dim):
    rows_per_w = total_rows // NW
    n_chunks = rows_per_w // CH
    cw = CH * dim
    mesh = plsc.VectorSubcoreMesh(core_axis_name="c", subcore_axis_name="s")

    @functools.partial(
        pl.kernel,
        mesh=mesh,
        out_type=jax.ShapeDtypeStruct((total_rows * dim,), jnp.float32),
        scratch_types=[
            [pltpu.VMEM((cw,), jnp.float32) for _ in range(NBUF)],
            [pltpu.VMEM((cw,), jnp.float32) for _ in range(NBUF)],
            [pltpu.VMEM((cw,), jnp.float32) for _ in range(NBUF)],
            [pltpu.SemaphoreType.DMA for _ in range(NBUF)],
            [pltpu.SemaphoreType.DMA for _ in range(NBUF)],
        ],
    )
    def k(in_hbm, pos_hbm, out_hbm, bufs, pbufs, obufs, in_sems, out_sems):
        wid = lax.axis_index("s") * NC + lax.axis_index("c")
        row0 = wid * rows_per_w
        pos0 = lax.rem(row0, seq)

        def start_in(g, b):
            r = (row0 + g * CH) * dim
            p = (pos0 + g * CH) * dim
            pltpu.async_copy(in_hbm.at[pl.ds(r, cw)], bufs[b], in_sems[b])
            pltpu.async_copy(pos_hbm.at[pl.ds(p, cw)], pbufs[b], in_sems[b])

        def wait_in(b):
            pltpu.make_async_copy(in_hbm.at[pl.ds(0, cw)], bufs[b], in_sems[b]).wait()
            pltpu.make_async_copy(pos_hbm.at[pl.ds(0, cw)], pbufs[b], in_sems[b]).wait()

        def start_out(g, b):
            r = (row0 + g * CH) * dim
            pltpu.async_copy(obufs[b], out_hbm.at[pl.ds(r, cw)], out_sems[b])

        def wait_out(b):
            pltpu.make_async_copy(obufs[b], out_hbm.at[pl.ds(0, cw)], out_sems[b]).wait()

        for b in range(NBUF):
            start_in(b, b)

        def step(g, b):
            wait_in(b)

            @pl.when(g >= NBUF)
            def _():
                wait_out(b)

            @plsc.parallel_loop(0, cw, step=L, unroll=8)
            def add_body(i):
                obufs[b][pl.ds(i, L)] = bufs[b][pl.ds(i, L)] + pbufs[b][pl.ds(i, L)]

            start_out(g, b)

            @pl.when(g + NBUF < n_chunks)
            def _():
                start_in(g + NBUF, b)

        def outer(go, carry):
            g = go * NBUF
            for b in range(NBUF):
                step(g + b, b)
            return carry

        lax.fori_loop(0, n_chunks // NBUF, outer, None)
        for b in range(NBUF):
            wait_out(b)

    return k


def _tc_add_kernel(x_ref, p_ref, o_ref):
    o_ref[0] = x_ref[0] + p_ref[...]


def _tc_add(x, pos_table):
    batch, seq, dim = x.shape
    grid = (seq // SEQ_BLOCK, batch)
    return pl.pallas_call(
        _tc_add_kernel,
        grid=grid,
        in_specs=[
            pl.BlockSpec((1, SEQ_BLOCK, dim), lambda s, b: (b, s, 0)),
            pl.BlockSpec((SEQ_BLOCK, dim), lambda s, b: (s, 0)),
        ],
        out_specs=pl.BlockSpec((1, SEQ_BLOCK, dim), lambda s, b: (b, s, 0)),
        out_shape=jax.ShapeDtypeStruct(x.shape, x.dtype),
    )(x, pos_table)


B_SC = 1


def kernel(inputs, pos_table):
    b, s, d = inputs.shape
    sc_flat = inputs[:B_SC].reshape(B_SC * s * d)
    out_sc = _make_sc(B_SC * s, s, d)(sc_flat, pos_table.reshape(s * d))
    out_tc = _tc_add(inputs[B_SC:], pos_table)
    return jnp.concatenate([out_sc.reshape(B_SC, s, d), out_tc], axis=0)


# speedup vs baseline: 5.4433x; 4.0318x over previous
import jax
import jax.numpy as jnp
from jax.experimental import pallas as pl

SEQ_BLOCK = 1024


def _add_kernel(x_ref, p_ref, o_ref):
    o_ref[0] = x_ref[0] + p_ref[...]


def kernel(inputs, pos_table):
    batch, seq, dim = inputs.shape
    grid = (seq // SEQ_BLOCK, batch)
    return pl.pallas_call(
        _add_kernel,
        grid=grid,
        in_specs=[
            pl.BlockSpec((1, SEQ_BLOCK, dim), lambda s, b: (b, s, 0)),
            pl.BlockSpec((SEQ_BLOCK, dim), lambda s, b: (s, 0)),
        ],
        out_specs=pl.BlockSpec((1, SEQ_BLOCK, dim), lambda s, b: (b, s, 0)),
        out_shape=jax.ShapeDtypeStruct(inputs.shape, inputs.dtype),
    )(inputs, pos_table)


# speedup vs baseline: 5.6700x; 1.0417x over previous
import jax
import jax.numpy as jnp
from jax.experimental import pallas as pl

SEQ_BLOCK = 2048


def _add_kernel(x_ref, p_ref, o_ref):
    o_ref[0] = x_ref[0] + p_ref[...]


def kernel(inputs, pos_table):
    batch, seq, dim = inputs.shape
    grid = (seq // SEQ_BLOCK, batch)
    return pl.pallas_call(
        _add_kernel,
        grid=grid,
        in_specs=[
            pl.BlockSpec((1, SEQ_BLOCK, dim), lambda s, b: (b, s, 0)),
            pl.BlockSpec((SEQ_BLOCK, dim), lambda s, b: (s, 0)),
        ],
        out_specs=pl.BlockSpec((1, SEQ_BLOCK, dim), lambda s, b: (b, s, 0)),
        out_shape=jax.ShapeDtypeStruct(inputs.shape, inputs.dtype),
    )(inputs, pos_table)
